# baseline (speedup 1.0000x reference)
import functools

import jax
import jax.numpy as jnp
from jax import lax
from jax.experimental import pallas as pl
from jax.experimental.pallas import tpu as pltpu
from jax.experimental.pallas import tpu_sc as plsc

NC = 2
NS = 16
NW = NC * NS
CH = 20480
UNROLL = 8


def _mm_body(w_ref, x_ref, o_ref):
    o_ref[...] = lax.dot_general(
        w_ref[...], x_ref[...], (((0,), (1,)), ((), ())),
        preferred_element_type=jnp.float32)


def _matmul_t(x, W):
    n, d = x.shape
    c = W.shape[1]
    return pl.pallas_call(
        _mm_body,
        out_shape=jax.ShapeDtypeStruct((c, n), jnp.float32),
    )(W, x)


def _tr_body(i_ref, o_ref):
    o_ref[...] = i_ref[...].T


def _transpose(outT):
    c, n = outT.shape
    return pl.pallas_call(
        _tr_body,
        out_shape=jax.ShapeDtypeStruct((n, c), jnp.float32),
    )(outT)


def _make_sc_agg(n, c, nch, shift):
    groups = CH // 16
    lowmask = (1 << shift) - 1

    def body(pk_hbm, ypk_hbm, b_hbm, out_hbm,
             ysp, acc0, acc1, pb0, pb1, bv, semA, semB):
        cid = lax.axis_index("c")
        sid = lax.axis_index("s")
        wid = sid * NC + cid
        col0 = wid * 2

        pltpu.async_copy(pk_hbm.at[pl.ds(0, CH)], pb0, semA)
        pltpu.sync_copy(ypk_hbm.at[wid], ysp)
        pltpu.sync_copy(b_hbm, bv)

        def zstep(i, _):
            acc0[pl.ds(i * 16, 16)] = jnp.zeros((16,), jnp.float32)
            acc1[pl.ds(i * 16, 16)] = jnp.zeros((16,), jnp.float32)
            return 0
        lax.fori_loop(0, (n + 16) // 16, zstep, 0)

        def process(pb):
            def gstep(k, _):
                offs = [(k * UNROLL + u) * 16 for u in range(UNROLL)]
                pk = [pb[pl.ds(o, 16)] for o in offs]
                ss = [jnp.right_shift(p, shift) for p in pk]
                dd = [jnp.bitwise_and(p, lowmask) for p in pk]
                gs = [plsc.load_gather(ysp, [s]) for s in ss]
                f0 = [plsc.bitcast(jnp.left_shift(g, 16), jnp.float32)
                      for g in gs]
                f1 = [plsc.bitcast(jnp.bitwise_and(g, -65536), jnp.float32)
                      for g in gs]
                for u in range(UNROLL):
                    plsc.addupdate_scatter(acc0, [dd[u]], f0[u])
                for u in range(UNROLL):
                    plsc.addupdate_scatter(acc1, [dd[u]], f1[u])
                return 0
            lax.fori_loop(0, groups // UNROLL, gstep, 0)

        def pair_step(p, _):
            ch0 = p * 2
            pltpu.async_copy(pk_hbm.at[pl.ds((ch0 + 1) * CH, CH)], pb1, semB)
            pltpu.make_async_copy(pk_hbm.at[pl.ds(ch0 * CH, CH)], pb0,
                                  semA).wait()
            process(pb0)
            pltpu.async_copy(pk_hbm.at[pl.ds((ch0 + 2) * CH, CH)], pb0, semA)
            pltpu.make_async_copy(pk_hbm.at[pl.ds((ch0 + 1) * CH, CH)], pb1,
                                  semB).wait()
            process(pb1)
            return 0
        lax.fori_loop(0, nch // 2, pair_step, 0)
        pltpu.make_async_copy(pk_hbm.at[pl.ds(nch * CH, CH)], pb0,
                              semA).wait()

        i16 = lax.iota(jnp.int32, 16)
        b0 = plsc.load_gather(bv, [i16 * 0 + col0])
        b1 = plsc.load_gather(bv, [i16 * 0 + col0 + 1])

        def fstep(i, _):
            sl = pl.ds(i * 16, 16)
            v = ysp[sl]
            y0 = plsc.bitcast(jnp.left_shift(v, 16), jnp.float32)
            y1 = plsc.bitcast(jnp.bitwise_and(v, -65536), jnp.float32)
            acc0[sl] = acc0[sl] + y0 + b0
            acc1[sl] = acc1[sl] + y1 + b1
            return 0
        lax.fori_loop(0, n // 16, fstep, 0)
        pltpu.sync_copy(acc0.at[pl.ds(0, n)], out_hbm.at[col0])
        pltpu.sync_copy(acc1.at[pl.ds(0, n)], out_hbm.at[col0 + 1])

    mesh = plsc.VectorSubcoreMesh(core_axis_name="c", subcore_axis_name="s")
    return pl.kernel(
        body,
        out_type=jax.ShapeDtypeStruct((c, n), jnp.float32),
        mesh=mesh,
        compiler_params=pltpu.CompilerParams(use_tc_tiling_on_sc=False,
                                             needs_layout_passes=False),
        scratch_types=[
            pltpu.VMEM((n,), jnp.int32),
            pltpu.VMEM((n + 16,), jnp.float32),
            pltpu.VMEM((n + 16,), jnp.float32),
            pltpu.VMEM((CH,), jnp.int32),
            pltpu.VMEM((CH,), jnp.int32),
            pltpu.VMEM((c,), jnp.float32),
            pltpu.SemaphoreType.DMA,
            pltpu.SemaphoreType.DMA,
        ],
    )


@jax.jit
def kernel(x, edge_index, W, b):
    n, d = x.shape
    c = W.shape[1]
    e = edge_index.shape[1]
    assert c == 2 * NW and n % 16 == 0

    shift = int(n).bit_length()
    assert 2 * shift <= 31

    nch = -(-e // CH)
    nch += nch % 2
    ep = nch * CH
    pad = ep + CH - e
    src = jnp.concatenate([edge_index[0], jnp.zeros((pad,), jnp.int32)])
    dst = jnp.concatenate([edge_index[1], jnp.full((pad,), n, jnp.int32)])
    pk = jnp.left_shift(src, shift) | dst

    yT = _matmul_t(x, W)
    ybits = lax.bitcast_convert_type(
        yT.astype(jnp.bfloat16), jnp.uint16).astype(jnp.uint32)
    ypk = lax.bitcast_convert_type(
        ybits[0::2] | (ybits[1::2] << 16), jnp.int32)

    outT = _make_sc_agg(n, c, nch, shift)(pk, ypk, b)
    return _transpose(outT)


# speedup vs baseline: 1.0110x; 1.0110x over previous
import functools

import jax
import jax.numpy as jnp
from jax import lax
from jax.experimental import pallas as pl
from jax.experimental.pallas import tpu as pltpu
from jax.experimental.pallas import tpu_sc as plsc

NC = 2
NS = 16
NW = NC * NS
CH = 8192
UNROLL = 8


def _mm_body(w_ref, x_ref, o_ref):
    o_ref[...] = lax.dot_general(
        w_ref[...], x_ref[...], (((0,), (1,)), ((), ())),
        preferred_element_type=jnp.float32)


def _matmul_t(x, W):
    n, d = x.shape
    c = W.shape[1]
    return pl.pallas_call(
        _mm_body,
        out_shape=jax.ShapeDtypeStruct((c, n), jnp.float32),
    )(W, x)


def _tr_body(i_ref, o_ref):
    o_ref[...] = i_ref[...].T


def _transpose(outT):
    c, n = outT.shape
    return pl.pallas_call(
        _tr_body,
        out_shape=jax.ShapeDtypeStruct((n, c), jnp.float32),
    )(outT)


def _make_sc_agg(n, c, nch, shift):
    groups = CH // 16
    lowmask = (1 << shift) - 1

    def body(pk_hbm, ypk_hbm, b_hbm, out_hbm,
             ysp, acc0, acc1, pb0, pb1, bv, semA, semB):
        cid = lax.axis_index("c")
        sid = lax.axis_index("s")
        wid = sid * NC + cid
        col0 = wid * 2

        def choff(q):
            return lax.rem(q + wid, nch) * CH

        pltpu.async_copy(pk_hbm.at[pl.ds(choff(0), CH)], pb0, semA)
        pltpu.sync_copy(ypk_hbm.at[wid], ysp)
        pltpu.sync_copy(b_hbm, bv)

        def zstep(i, _):
            acc0[pl.ds(i * 16, 16)] = jnp.zeros((16,), jnp.float32)
            acc1[pl.ds(i * 16, 16)] = jnp.zeros((16,), jnp.float32)
            return 0
        lax.fori_loop(0, (n + 16) // 16, zstep, 0)

        def process(pb):
            def gstep(k, _):
                offs = [(k * UNROLL + u) * 16 for u in range(UNROLL)]
                pk = [pb[pl.ds(o, 16)] for o in offs]
                ss = [jnp.right_shift(p, shift) for p in pk]
                dd = [jnp.bitwise_and(p, lowmask) for p in pk]
                gs = [plsc.load_gather(ysp, [s]) for s in ss]
                f0 = [plsc.bitcast(jnp.left_shift(g, 16), jnp.float32)
                      for g in gs]
                f1 = [plsc.bitcast(jnp.bitwise_and(g, -65536), jnp.float32)
                      for g in gs]
                for u in range(UNROLL):
                    plsc.addupdate_scatter(acc0, [dd[u]], f0[u])
                for u in range(UNROLL):
                    plsc.addupdate_scatter(acc1, [dd[u]], f1[u])
                return 0
            lax.fori_loop(0, groups // UNROLL, gstep, 0)

        def pair_step(p, _):
            ch0 = p * 2
            pltpu.async_copy(pk_hbm.at[pl.ds(choff(ch0 + 1), CH)], pb1, semB)
            pltpu.make_async_copy(pk_hbm.at[pl.ds(choff(ch0), CH)], pb0,
                                  semA).wait()
            process(pb0)
            pltpu.async_copy(pk_hbm.at[pl.ds(choff(ch0 + 2), CH)], pb0, semA)
            pltpu.make_async_copy(pk_hbm.at[pl.ds(choff(ch0 + 1), CH)], pb1,
                                  semB).wait()
            process(pb1)
            return 0
        lax.fori_loop(0, nch // 2, pair_step, 0)
        pltpu.make_async_copy(pk_hbm.at[pl.ds(choff(nch), CH)], pb0,
                              semA).wait()

        i16 = lax.iota(jnp.int32, 16)
        b0 = plsc.load_gather(bv, [i16 * 0 + col0])
        b1 = plsc.load_gather(bv, [i16 * 0 + col0 + 1])

        def fstep(i, _):
            sl = pl.ds(i * 16, 16)
            v = ysp[sl]
            y0 = plsc.bitcast(jnp.left_shift(v, 16), jnp.float32)
            y1 = plsc.bitcast(jnp.bitwise_and(v, -65536), jnp.float32)
            acc0[sl] = acc0[sl] + y0 + b0
            acc1[sl] = acc1[sl] + y1 + b1
            return 0
        lax.fori_loop(0, n // 16, fstep, 0)
        pltpu.sync_copy(acc0.at[pl.ds(0, n)], out_hbm.at[col0])
        pltpu.sync_copy(acc1.at[pl.ds(0, n)], out_hbm.at[col0 + 1])

    mesh = plsc.VectorSubcoreMesh(core_axis_name="c", subcore_axis_name="s")
    return pl.kernel(
        body,
        out_type=jax.ShapeDtypeStruct((c, n), jnp.float32),
        mesh=mesh,
        compiler_params=pltpu.CompilerParams(use_tc_tiling_on_sc=False,
                                             needs_layout_passes=False),
        scratch_types=[
            pltpu.VMEM((n,), jnp.int32),
            pltpu.VMEM((n + 16,), jnp.float32),
            pltpu.VMEM((n + 16,), jnp.float32),
            pltpu.VMEM((CH,), jnp.int32),
            pltpu.VMEM((CH,), jnp.int32),
            pltpu.VMEM((c,), jnp.float32),
            pltpu.SemaphoreType.DMA,
            pltpu.SemaphoreType.DMA,
        ],
    )


@jax.jit
def kernel(x, edge_index, W, b):
    n, d = x.shape
    c = W.shape[1]
    e = edge_index.shape[1]
    assert c == 2 * NW and n % 16 == 0

    shift = int(n).bit_length()
    assert 2 * shift <= 31

    nch = -(-e // CH)
    nch += nch % 2
    ep = nch * CH
    pad = ep - e
    src = jnp.concatenate([edge_index[0], jnp.zeros((pad,), jnp.int32)])
    dst = jnp.concatenate([edge_index[1], jnp.full((pad,), n, jnp.int32)])
    pk = jnp.left_shift(src, shift) | dst

    yT = _matmul_t(x, W)
    ybits = lax.bitcast_convert_type(
        yT.astype(jnp.bfloat16), jnp.uint16).astype(jnp.uint32)
    ypk = lax.bitcast_convert_type(
        ybits[0::2] | (ybits[1::2] << 16), jnp.int32)

    outT = _make_sc_agg(n, c, nch, shift)(pk, ypk, b)
    return _transpose(outT)
